# baseline (device time: 32165 ns/iter reference)
import jax
import jax.numpy as jnp
from jax import lax
from jax.experimental import pallas as pl
from jax.experimental.pallas import tpu as pltpu

N_DEV = 32
V_PER = 4096
N_IDX = 1024
D = 512
B = N_IDX // N_DEV
CHUNK = 2048
SCALE = 32.0

OFFSETS = sorted(range(1, N_DEV), key=lambda d: -min(d, N_DEV - d))


def kernel(table, idx):
    def body(table_ref, idx_ref, out_ref, qpart, gq, qout,
             send1, recv1, send2, recv2):
        me = lax.axis_index("i")

        barrier_sem = pltpu.get_barrier_semaphore()
        for d in range(1, N_DEV):
            pl.semaphore_signal(
                barrier_sem, inc=1,
                device_id=((me + d) % N_DEV,),
                device_id_type=pl.DeviceIdType.MESH,
            )

        local = idx_ref[:] - me * V_PER
        local2d = local.reshape(N_IDX, 1)
        acc = jnp.zeros((N_IDX, D), jnp.float32)
        for j in range(V_PER // CHUNK):
            cols = lax.broadcasted_iota(jnp.int32, (N_IDX, CHUNK), 1) + j * CHUNK
            onehot = (cols == local2d).astype(jnp.bfloat16)
            t_chunk = table_ref[j * CHUNK:(j + 1) * CHUNK, :].astype(jnp.bfloat16)
            acc = acc + jnp.dot(onehot, t_chunk,
                                preferred_element_type=jnp.float32)
        qpart[...] = jnp.clip(
            jnp.round(acc * SCALE), -127.0, 127.0).astype(jnp.int8)

        pl.semaphore_wait(barrier_sem, N_DEV - 1)

        p1 = []
        for d in OFFSETS:
            p = (me + d) % N_DEV
            rdma = pltpu.make_async_remote_copy(
                src_ref=qpart.at[pl.ds(p * B, B), :],
                dst_ref=gq.at[pl.ds(d * B, B), :],
                send_sem=send1.at[d],
                recv_sem=recv1.at[d],
                device_id=(p,),
                device_id_type=pl.DeviceIdType.MESH,
            )
            rdma.start()
            p1.append(rdma)

        gq[pl.ds(0, B), :] = qpart[pl.ds(me * B, B), :]
        for r in p1:
            r.wait_recv()
        rows32 = lax.broadcasted_iota(jnp.int32, (B, N_DEV * B), 0)
        cols32 = lax.broadcasted_iota(jnp.int32, (B, N_DEV * B), 1)
        sel = (cols32 % B == rows32).astype(jnp.bfloat16)
        blk = jnp.dot(sel, gq[...].astype(jnp.bfloat16),
                      preferred_element_type=jnp.float32)
        qpart[pl.ds(me * B, B), :] = blk.astype(jnp.int8)

        p2 = []
        for d in OFFSETS:
            rdma = pltpu.make_async_remote_copy(
                src_ref=qpart.at[pl.ds(me * B, B), :],
                dst_ref=qout.at[pl.ds(me * B, B), :],
                send_sem=send2.at[d],
                recv_sem=recv2.at[d],
                device_id=((me + d) % N_DEV,),
                device_id_type=pl.DeviceIdType.MESH,
            )
            rdma.start()
            p2.append(rdma)

        qout[pl.ds(me * B, B), :] = qpart[pl.ds(me * B, B), :]
        for r in p1:
            r.wait_send()
        for r in p2:
            r.wait()
        out_ref[...] = qout[...].astype(jnp.bfloat16) * jnp.bfloat16(1.0 / SCALE)

    return pl.pallas_call(
        body,
        out_shape=jax.ShapeDtypeStruct((N_IDX, D), jnp.bfloat16),
        in_specs=[
            pl.BlockSpec(memory_space=pltpu.VMEM),
            pl.BlockSpec(memory_space=pltpu.VMEM),
        ],
        out_specs=pl.BlockSpec(memory_space=pltpu.VMEM),
        scratch_shapes=[
            pltpu.VMEM((N_IDX, D), jnp.int8),
            pltpu.VMEM((N_IDX, D), jnp.int8),
            pltpu.VMEM((N_IDX, D), jnp.int8),
            pltpu.SemaphoreType.DMA((N_DEV,)),
            pltpu.SemaphoreType.DMA((N_DEV,)),
            pltpu.SemaphoreType.DMA((N_DEV,)),
            pltpu.SemaphoreType.DMA((N_DEV,)),
        ],
        compiler_params=pltpu.CompilerParams(collective_id=0),
    )(table, idx)


# device time: 31441 ns/iter; 1.0230x vs baseline; 1.0230x over previous
import jax
import jax.numpy as jnp
from jax import lax
from jax.experimental import pallas as pl
from jax.experimental.pallas import tpu as pltpu

N_DEV = 32
V_PER = 4096
N_IDX = 1024
D = 512
B = N_IDX // N_DEV
CHUNK = 2048
SCALE = 32.0


def kernel(table, idx):
    def body(table_ref, idx_ref, out_ref, qpart, gq, qout,
             send1, recv1, send2, recv2):
        me = lax.axis_index("i")

        barrier_sem = pltpu.get_barrier_semaphore()
        for d in range(1, N_DEV):
            pl.semaphore_signal(
                barrier_sem, inc=1,
                device_id=((me + d) % N_DEV,),
                device_id_type=pl.DeviceIdType.MESH,
            )

        local = idx_ref[:] - me * V_PER
        local2d = local.reshape(N_IDX, 1)
        acc = jnp.zeros((N_IDX, D), jnp.float32)
        for j in range(V_PER // CHUNK):
            cols = lax.broadcasted_iota(jnp.int32, (N_IDX, CHUNK), 1) + j * CHUNK
            onehot = (cols == local2d).astype(jnp.bfloat16)
            t_chunk = table_ref[j * CHUNK:(j + 1) * CHUNK, :].astype(jnp.bfloat16)
            acc = acc + jnp.dot(onehot, t_chunk,
                                preferred_element_type=jnp.float32)
        qpart[...] = jnp.clip(
            jnp.round(acc * SCALE), -127.0, 127.0).astype(jnp.int8)

        pl.semaphore_wait(barrier_sem, N_DEV - 1)

        p1 = []
        for d in range(1, N_DEV):
            p = (me + d) % N_DEV
            rdma = pltpu.make_async_remote_copy(
                src_ref=qpart.at[pl.ds(p * B, B), :],
                dst_ref=gq.at[pl.ds(d * B, B), :],
                send_sem=send1.at[d],
                recv_sem=recv1.at[d],
                device_id=(p,),
                device_id_type=pl.DeviceIdType.MESH,
            )
            rdma.start()
            p1.append(rdma)

        gq[pl.ds(0, B), :] = qpart[pl.ds(me * B, B), :]
        for d in range(1, N_DEV):
            p1[d - 1].wait_recv()
        rows32 = lax.broadcasted_iota(jnp.int32, (B, N_DEV * B), 0)
        cols32 = lax.broadcasted_iota(jnp.int32, (B, N_DEV * B), 1)
        sel = (cols32 % B == rows32).astype(jnp.bfloat16)
        blk = jnp.dot(sel, gq[...].astype(jnp.bfloat16),
                      preferred_element_type=jnp.float32)
        qpart[pl.ds(me * B, B), :] = blk.astype(jnp.int8)

        p2 = []
        for d in range(1, N_DEV):
            rdma = pltpu.make_async_remote_copy(
                src_ref=qpart.at[pl.ds(me * B, B), :],
                dst_ref=qout.at[pl.ds(me * B, B), :],
                send_sem=send2.at[d],
                recv_sem=recv2.at[d],
                device_id=((me + d) % N_DEV,),
                device_id_type=pl.DeviceIdType.MESH,
            )
            rdma.start()
            p2.append(rdma)

        qout[pl.ds(me * B, B), :] = qpart[pl.ds(me * B, B), :]
        for d in range(1, N_DEV):
            p1[d - 1].wait_send()
        for d in range(1, N_DEV):
            p2[d - 1].wait()
        out_ref[...] = qout[...].astype(jnp.bfloat16) * jnp.bfloat16(1.0 / SCALE)

    return pl.pallas_call(
        body,
        out_shape=jax.ShapeDtypeStruct((N_IDX, D), jnp.bfloat16),
        in_specs=[
            pl.BlockSpec(memory_space=pltpu.VMEM),
            pl.BlockSpec(memory_space=pltpu.VMEM),
        ],
        out_specs=pl.BlockSpec(memory_space=pltpu.VMEM),
        scratch_shapes=[
            pltpu.VMEM((N_IDX, D), jnp.int8),
            pltpu.VMEM((N_IDX, D), jnp.int8),
            pltpu.VMEM((N_IDX, D), jnp.int8),
            pltpu.SemaphoreType.DMA((N_DEV,)),
            pltpu.SemaphoreType.DMA((N_DEV,)),
            pltpu.SemaphoreType.DMA((N_DEV,)),
            pltpu.SemaphoreType.DMA((N_DEV,)),
        ],
        compiler_params=pltpu.CompilerParams(collective_id=0),
    )(table, idx)
